# baseline (speedup 1.0000x reference)
import jax
import jax.numpy as jnp
from jax.experimental import pallas as pl

N = 200000
B = 4
C = 64
G = 32


def _silu(x):
    return x * jax.nn.sigmoid(x)


def _gn(feats, batch_idx, gamma, beta):
    Cg = C // G
    g = feats.reshape(N, G, Cg)
    cnt = jax.ops.segment_sum(jnp.ones((N,), feats.dtype), batch_idx, num_segments=B)
    denom = cnt[:, None] * Cg
    s = jax.ops.segment_sum(g.sum(-1), batch_idx, num_segments=B)
    ss = jax.ops.segment_sum((g * g).sum(-1), batch_idx, num_segments=B)
    mean = s / denom
    var = ss / denom - mean * mean
    m = mean[batch_idx][:, :, None]
    v = var[batch_idx][:, :, None]
    out = (g - m) * jax.lax.rsqrt(v + 1e-5)
    return out.reshape(N, C) * gamma[None, :] + beta[None, :]


def _conv(feats, nbrs, W, b):
    padded = jnp.concatenate([feats, jnp.zeros((1, feats.shape[1]), feats.dtype)], axis=0)
    out = jnp.zeros((feats.shape[0], W.shape[2]), feats.dtype) + b[None, :]
    for k in range(27):
        out = out + padded[nbrs[k]] @ W[k]
    return out


def _residual_add_kernel(a_ref, b_ref, o_ref):
    o_ref[...] = a_ref[...] + b_ref[...]


def kernel(feats, emb, gamma1, beta1, W1, b1c, We, be, gamma2, beta2, W2, b2c, batch_idx, nbrs):
    h = _gn(feats, batch_idx, gamma1, beta1)
    h = _silu(h)
    h = _conv(h, nbrs, W1, b1c)
    emb_out = _silu(emb) @ We + be[None, :]
    h = h + emb_out[batch_idx]
    h = _gn(h, batch_idx, gamma2, beta2)
    h = _silu(h)
    h = _conv(h, nbrs, W2, b2c)
    T = 1000
    return pl.pallas_call(
        _residual_add_kernel,
        out_shape=jax.ShapeDtypeStruct((N, C), feats.dtype),
        grid=(N // T,),
        in_specs=[
            pl.BlockSpec((T, C), lambda i: (i, 0)),
            pl.BlockSpec((T, C), lambda i: (i, 0)),
        ],
        out_specs=pl.BlockSpec((T, C), lambda i: (i, 0)),
    )(feats, h)


# speedup vs baseline: 11.3811x; 11.3811x over previous
import functools

import jax
import jax.numpy as jnp
from jax import lax
from jax.experimental import pallas as pl
from jax.experimental.pallas import tpu as pltpu
from jax.experimental.pallas import tpu_sc as plsc

N = 200000
B = 4
C = 64
G = 32
K = 27
NPER = 50000
T = 1000
NT = N // T
TPB = NPER // T
NPAD = N + T
WR = 400
NWIN = N // WR
NWORK = 32
SENT = N

_f32 = jnp.float32


def _row4(ref, bidx):
    r = ref[0:1, :]
    for b in range(1, B):
        r = jnp.where(bidx == b, ref[b:b + 1, :], r)
    return r


def _stats1_body(x_ref, emb_ref, we_ref, bevec_ref, s_ref, ss_ref, eo_ref):
    t = pl.program_id(0)
    bidx = t // TPB
    x = x_ref[...]

    @pl.when(t == 0)
    def _():
        s_ref[...] = jnp.zeros_like(s_ref)
        ss_ref[...] = jnp.zeros_like(ss_ref)
        e = emb_ref[...]
        he = e * jax.nn.sigmoid(e)
        eo_ref[...] = (
            lax.dot_general(he, we_ref[...], (((1,), (0,)), ((), ())),
                            precision=lax.Precision.HIGHEST,
                            preferred_element_type=_f32)
            + bevec_ref[...]
        )

    oh = (lax.broadcasted_iota(jnp.int32, (8, 1), 0) == bidx).astype(_f32)
    s_ref[...] += oh * jnp.sum(x, axis=0)[None, :]
    ss_ref[...] += oh * jnp.sum(x * x, axis=0)[None, :]


def _stats2_body(x_ref, eo_ref, s_ref, ss_ref):
    t = pl.program_id(0)
    bidx = t // TPB
    x = x_ref[...] + _row4(eo_ref, bidx)

    @pl.when(t == 0)
    def _():
        s_ref[...] = jnp.zeros_like(s_ref)
        ss_ref[...] = jnp.zeros_like(ss_ref)

    oh = (lax.broadcasted_iota(jnp.int32, (8, 1), 0) == bidx).astype(_f32)
    s_ref[...] += oh * jnp.sum(x, axis=0)[None, :]
    ss_ref[...] += oh * jnp.sum(x * x, axis=0)[None, :]


def _mm_body(x_ref, a_ref, b_ref, w_ref, cb_ref, y_ref, *, add_emb, eo_ref=None):
    t = pl.program_id(0)
    bidx = jnp.minimum(t, NT - 1) // TPB
    x = x_ref[...]
    if add_emb:
        x = x + _row4(eo_ref, bidx)
    hp = x * _row4(a_ref, bidx) + _row4(b_ref, bidx)
    h = hp * jax.nn.sigmoid(hp)
    live = (t < NT).astype(_f32)
    h = h * live
    for k in range(K):
        yk = lax.dot_general(h, w_ref[k], (((1,), (0,)), ((), ())),
                             precision=lax.Precision.HIGHEST,
                             preferred_element_type=_f32)
        if k == 13:
            yk = yk + cb_ref[0:1, :] * live
        y_ref[k] = yk


def _mm_body_noemb(x_ref, a_ref, b_ref, w_ref, cb_ref, y_ref):
    _mm_body(x_ref, a_ref, b_ref, w_ref, cb_ref, y_ref, add_emb=False)


def _mm_body_emb(x_ref, eo_ref, a_ref, b_ref, w_ref, cb_ref, y_ref):
    _mm_body(x_ref, a_ref, b_ref, w_ref, cb_ref, y_ref, add_emb=True,
             eo_ref=eo_ref)


_VEC8 = pl.BlockSpec((8, C), lambda t: (0, 0))


def _stats1(feats, emb8, We, bevec):
    return pl.pallas_call(
        _stats1_body,
        grid=(NT,),
        in_specs=[
            pl.BlockSpec((T, C), lambda t: (t, 0)),
            pl.BlockSpec((8, 512), lambda t: (0, 0)),
            pl.BlockSpec((512, C), lambda t: (0, 0)),
            _VEC8,
        ],
        out_specs=[_VEC8, _VEC8, _VEC8],
        out_shape=[jax.ShapeDtypeStruct((8, C), _f32)] * 3,
    )(feats, emb8, We, bevec)


def _stats2(h1s, eo):
    return pl.pallas_call(
        _stats2_body,
        grid=(NT,),
        in_specs=[pl.BlockSpec((T, C), lambda t: (t, 0)), _VEC8],
        out_specs=[_VEC8, _VEC8],
        out_shape=[jax.ShapeDtypeStruct((8, C), _f32)] * 2,
    )(h1s, eo)


def _mm27(x, a, b, W, cbvec, eo=None):
    body = _mm_body_noemb if eo is None else _mm_body_emb
    xs = [x] if eo is None else [x, eo]
    in_specs = [pl.BlockSpec((T, C), lambda t: (jnp.minimum(t, NT - 1), 0))]
    if eo is not None:
        in_specs.append(_VEC8)
    in_specs += [
        _VEC8, _VEC8,
        pl.BlockSpec((K, C, C), lambda t: (0, 0, 0)),
        _VEC8,
    ]
    y = pl.pallas_call(
        body,
        grid=(NT + 1,),
        in_specs=in_specs,
        out_specs=pl.BlockSpec((K, T, C), lambda t: (0, t, 0)),
        out_shape=jax.ShapeDtypeStruct((K, NPAD, C), _f32),
    )(*xs, a, b, W, cbvec)
    return y.reshape(K * NPAD, C)


def _sc_gather_sum(yflat, nbrs3, resid):
    mesh = plsc.VectorSubcoreMesh(core_axis_name="c", subcore_axis_name="s")
    with_resid = resid is not None

    @functools.partial(
        pl.kernel,
        out_type=jax.ShapeDtypeStruct((N, C), _f32),
        mesh=mesh,
        compiler_params=pltpu.CompilerParams(use_tc_tiling_on_sc=False),
        scratch_types=[
            pltpu.VMEM((WR,), jnp.int32),
            pltpu.VMEM((WR, C), _f32),
            pltpu.VMEM((WR, C), _f32),
            pltpu.SemaphoreType.DMA,
        ],
    )
    def k(*refs):
        if with_resid:
            y_hbm, n_hbm, r_hbm, o_hbm, idx_v, g_v, acc_v, sem = refs
        else:
            y_hbm, n_hbm, o_hbm, idx_v, g_v, acc_v, sem = refs
        wid = lax.axis_index("s") * 2 + lax.axis_index("c")

        @pl.loop(0, NWIN // NWORK + 1)
        def _(it):
            w = wid + NWORK * it

            @pl.when(w < NWIN)
            def _():
                base = w * WR
                if with_resid:
                    pltpu.async_copy(r_hbm.at[pl.ds(base, WR)], acc_v, sem).wait()
                else:
                    pltpu.async_copy(y_hbm.at[pl.ds(SENT, WR)], acc_v, sem).wait()

                @pl.loop(0, K)
                def _(kk):
                    pltpu.async_copy(n_hbm.at[kk, w], idx_v, sem).wait()
                    off = kk * NPAD

                    @pl.loop(0, WR // 16)
                    def _(j):
                        sl = pl.ds(j * 16, 16)
                        idx_v[sl] = jnp.minimum(idx_v[sl], SENT) + off

                    pltpu.async_copy(y_hbm.at[idx_v], g_v, sem).wait()

                    @pl.loop(0, WR)
                    def _(r):
                        for cc in range(C // 16):
                            sl = pl.ds(cc * 16, 16)
                            acc_v[r, sl] = acc_v[r, sl] + g_v[r, sl]

                pltpu.async_copy(acc_v, o_hbm.at[pl.ds(base, WR)], sem).wait()

    args = (yflat, nbrs3, resid) if with_resid else (yflat, nbrs3)
    return k(*args)


def _affine(s, ss, gamma, beta):
    cnt = float(NPER * (C // G))
    sg = s[:B].reshape(B, G, C // G).sum(-1)
    ssg = ss[:B].reshape(B, G, C // G).sum(-1)
    mean = sg / cnt
    var = ssg / cnt - mean * mean
    rstd = lax.rsqrt(var + 1e-5)
    meanc = jnp.repeat(mean, C // G, axis=-1)
    rstdc = jnp.repeat(rstd, C // G, axis=-1)
    a = rstdc * gamma[None, :]
    b = beta[None, :] - meanc * a
    pad = jnp.zeros((8 - B, C), _f32)
    return jnp.concatenate([a, pad], 0), jnp.concatenate([b, pad], 0)


def kernel(feats, emb, gamma1, beta1, W1, b1c, We, be, gamma2, beta2, W2, b2c,
           batch_idx, nbrs):
    emb8 = jnp.concatenate([emb, jnp.zeros((8 - B, emb.shape[1]), _f32)], 0)
    bevec = jnp.broadcast_to(be[None, :], (8, C))
    b1vec = jnp.broadcast_to(b1c[None, :], (8, C))
    b2vec = jnp.broadcast_to(b2c[None, :], (8, C))
    nbrs3 = nbrs.reshape(K, NWIN, WR)

    s0, ss0, eo = _stats1(feats, emb8, We, bevec)
    a1, b1 = _affine(s0, ss0, gamma1, beta1)
    y1 = _mm27(feats, a1, b1, W1, b1vec)
    h1s = _sc_gather_sum(y1, nbrs3, None)

    s1, ss1 = _stats2(h1s, eo)
    a2, b2 = _affine(s1, ss1, gamma2, beta2)
    y2 = _mm27(h1s, a2, b2, W2, b2vec, eo=eo)
    out = _sc_gather_sum(y2, nbrs3, feats)
    return out


# speedup vs baseline: 12.4975x; 1.0981x over previous
import functools

import jax
import jax.numpy as jnp
from jax import lax
from jax.experimental import pallas as pl
from jax.experimental.pallas import tpu as pltpu
from jax.experimental.pallas import tpu_sc as plsc

N = 200000
B = 4
C = 64
G = 32
K = 27
NPER = 50000
T = 1000
NT = N // T
TPB = NPER // T
NPAD = N + T
WR = 400
NWIN = N // WR
NWORK = 32
SENT = N

_f32 = jnp.float32


def _row4(ref, bidx):
    r = ref[0:1, :]
    for b in range(1, B):
        r = jnp.where(bidx == b, ref[b:b + 1, :], r)
    return r


def _stats1_body(x_ref, emb_ref, we_ref, bevec_ref, s_ref, ss_ref, eo_ref):
    t = pl.program_id(0)
    bidx = t // TPB
    x = x_ref[...]

    @pl.when(t == 0)
    def _():
        s_ref[...] = jnp.zeros_like(s_ref)
        ss_ref[...] = jnp.zeros_like(ss_ref)
        e = emb_ref[...]
        he = e * jax.nn.sigmoid(e)
        eo_ref[...] = (
            lax.dot_general(he, we_ref[...], (((1,), (0,)), ((), ())),
                            precision=lax.Precision.HIGHEST,
                            preferred_element_type=_f32)
            + bevec_ref[...]
        )

    oh = (lax.broadcasted_iota(jnp.int32, (8, 1), 0) == bidx).astype(_f32)
    s_ref[...] += oh * jnp.sum(x, axis=0)[None, :]
    ss_ref[...] += oh * jnp.sum(x * x, axis=0)[None, :]


def _stats2_body(x_ref, eo_ref, s_ref, ss_ref):
    t = pl.program_id(0)
    bidx = t // TPB
    x = x_ref[...] + _row4(eo_ref, bidx)

    @pl.when(t == 0)
    def _():
        s_ref[...] = jnp.zeros_like(s_ref)
        ss_ref[...] = jnp.zeros_like(ss_ref)

    oh = (lax.broadcasted_iota(jnp.int32, (8, 1), 0) == bidx).astype(_f32)
    s_ref[...] += oh * jnp.sum(x, axis=0)[None, :]
    ss_ref[...] += oh * jnp.sum(x * x, axis=0)[None, :]


def _mm_body(x_ref, a_ref, b_ref, w_ref, cb_ref, y_ref, *, add_emb, eo_ref=None):
    t = pl.program_id(0)
    bidx = jnp.minimum(t, NT - 1) // TPB
    x = x_ref[...]
    if add_emb:
        x = x + _row4(eo_ref, bidx)
    hp = x * _row4(a_ref, bidx) + _row4(b_ref, bidx)
    h = hp * jax.nn.sigmoid(hp)
    live = (t < NT).astype(_f32)
    h = h * live
    for k in range(K):
        yk = lax.dot_general(h, w_ref[k], (((1,), (0,)), ((), ())),
                             precision=lax.Precision.HIGHEST,
                             preferred_element_type=_f32)
        if k == 13:
            yk = yk + cb_ref[0:1, :] * live
        y_ref[k] = yk


def _mm_body_noemb(x_ref, a_ref, b_ref, w_ref, cb_ref, y_ref):
    _mm_body(x_ref, a_ref, b_ref, w_ref, cb_ref, y_ref, add_emb=False)


def _mm_body_emb(x_ref, eo_ref, a_ref, b_ref, w_ref, cb_ref, y_ref):
    _mm_body(x_ref, a_ref, b_ref, w_ref, cb_ref, y_ref, add_emb=True,
             eo_ref=eo_ref)


_VEC8 = pl.BlockSpec((8, C), lambda t: (0, 0))


def _stats1(feats, emb8, We, bevec):
    return pl.pallas_call(
        _stats1_body,
        grid=(NT,),
        in_specs=[
            pl.BlockSpec((T, C), lambda t: (t, 0)),
            pl.BlockSpec((8, 512), lambda t: (0, 0)),
            pl.BlockSpec((512, C), lambda t: (0, 0)),
            _VEC8,
        ],
        out_specs=[_VEC8, _VEC8, _VEC8],
        out_shape=[jax.ShapeDtypeStruct((8, C), _f32)] * 3,
    )(feats, emb8, We, bevec)


def _stats2(h1s, eo):
    return pl.pallas_call(
        _stats2_body,
        grid=(NT,),
        in_specs=[pl.BlockSpec((T, C), lambda t: (t, 0)), _VEC8],
        out_specs=[_VEC8, _VEC8],
        out_shape=[jax.ShapeDtypeStruct((8, C), _f32)] * 2,
    )(h1s, eo)


def _mm27(x, a, b, W, cbvec, eo=None):
    body = _mm_body_noemb if eo is None else _mm_body_emb
    xs = [x] if eo is None else [x, eo]
    in_specs = [pl.BlockSpec((T, C), lambda t: (jnp.minimum(t, NT - 1), 0))]
    if eo is not None:
        in_specs.append(_VEC8)
    in_specs += [
        _VEC8, _VEC8,
        pl.BlockSpec((K, C, C), lambda t: (0, 0, 0)),
        _VEC8,
    ]
    y = pl.pallas_call(
        body,
        grid=(NT + 1,),
        in_specs=in_specs,
        out_specs=pl.BlockSpec((K, T, C), lambda t: (0, t, 0)),
        out_shape=jax.ShapeDtypeStruct((K, NPAD, C), _f32),
    )(*xs, a, b, W, cbvec)
    return y.reshape(K * NPAD, C)


def _sc_gather_sum(yflat, idxw, resid):
    mesh = plsc.VectorSubcoreMesh(core_axis_name="c", subcore_axis_name="s")
    with_resid = resid is not None

    @functools.partial(
        pl.kernel,
        out_type=jax.ShapeDtypeStruct((N, C), _f32),
        mesh=mesh,
        compiler_params=pltpu.CompilerParams(use_tc_tiling_on_sc=False),
        scratch_types=[
            pltpu.VMEM((K, WR), jnp.int32),
            pltpu.VMEM((WR,), jnp.int32),
            pltpu.VMEM((WR, C), _f32),
            pltpu.VMEM((WR, C), _f32),
            pltpu.VMEM((WR, C), _f32),
            pltpu.VMEM_SHARED((16 * WR, C), _f32),
            pltpu.SemaphoreType.DMA,
            pltpu.SemaphoreType.DMA,
            pltpu.SemaphoreType.DMA,
        ],
    )
    def k(*refs):
        if with_resid:
            y_hbm, i_hbm, r_hbm, o_hbm = refs[:4]
        else:
            y_hbm, i_hbm, o_hbm = refs[:3]
        idxw_v, idv_v, z_v, ga_v, gb_v, acc_sh, sem_a, sem_b, sem_m = refs[-9:]
        cid = lax.axis_index("c")
        sid = lax.axis_index("s")
        wid = sid * 2 + cid

        @pl.loop(0, WR // 16)
        def _(j):
            sl = pl.ds(j * 16, 16)
            idv_v[sl] = jnp.arange(16, dtype=jnp.int32) + (j * 16 + sid * WR)

        @pl.loop(0, WR)
        def _(r):
            for cc in range(C // 16):
                z_v[r, pl.ds(cc * 16, 16)] = jnp.zeros((16,), _f32)

        def gather(kk, buf, sem):
            return pltpu.async_copy(y_hbm.at[idxw_v.at[kk]], buf, sem)

        def scat_add(buf):
            pltpu.sync_copy(buf, acc_sh.at[idv_v], add=True)

        @pl.loop(0, NWIN // NWORK + 1)
        def _(it):
            w = wid + NWORK * it

            @pl.when(w < NWIN)
            def _():
                base = w * WR
                pltpu.async_copy(i_hbm.at[w], idxw_v, sem_m).wait()
                if with_resid:
                    pltpu.async_copy(
                        r_hbm.at[pl.ds(base, WR)],
                        acc_sh.at[pl.ds(sid * WR, WR)], sem_m).wait()
                else:
                    pltpu.async_copy(
                        z_v, acc_sh.at[pl.ds(sid * WR, WR)], sem_m).wait()
                gather(0, ga_v, sem_a)

                @pl.loop(0, (K - 1) // 2)
                def _(j):
                    gather(2 * j + 1, gb_v, sem_b)
                    pltpu.make_async_copy(y_hbm.at[idxw_v.at[0]], ga_v,
                                          sem_a).wait()
                    scat_add(ga_v)
                    gather(2 * j + 2, ga_v, sem_a)
                    pltpu.make_async_copy(y_hbm.at[idxw_v.at[0]], gb_v,
                                          sem_b).wait()
                    scat_add(gb_v)

                pltpu.make_async_copy(y_hbm.at[idxw_v.at[0]], ga_v,
                                      sem_a).wait()
                scat_add(ga_v)
                pltpu.async_copy(acc_sh.at[pl.ds(sid * WR, WR)],
                                 o_hbm.at[pl.ds(base, WR)], sem_m).wait()

    args = (yflat, idxw, resid) if with_resid else (yflat, idxw)
    return k(*args)


def _affine(s, ss, gamma, beta):
    cnt = float(NPER * (C // G))
    sg = s[:B].reshape(B, G, C // G).sum(-1)
    ssg = ss[:B].reshape(B, G, C // G).sum(-1)
    mean = sg / cnt
    var = ssg / cnt - mean * mean
    rstd = lax.rsqrt(var + 1e-5)
    meanc = jnp.repeat(mean, C // G, axis=-1)
    rstdc = jnp.repeat(rstd, C // G, axis=-1)
    a = rstdc * gamma[None, :]
    b = beta[None, :] - meanc * a
    pad = jnp.zeros((8 - B, C), _f32)
    return jnp.concatenate([a, pad], 0), jnp.concatenate([b, pad], 0)


def kernel(feats, emb, gamma1, beta1, W1, b1c, We, be, gamma2, beta2, W2, b2c,
           batch_idx, nbrs):
    emb8 = jnp.concatenate([emb, jnp.zeros((8 - B, emb.shape[1]), _f32)], 0)
    bevec = jnp.broadcast_to(be[None, :], (8, C))
    b1vec = jnp.broadcast_to(b1c[None, :], (8, C))
    b2vec = jnp.broadcast_to(b2c[None, :], (8, C))
    idxw = (jnp.minimum(nbrs, SENT)
            + (jnp.arange(K, dtype=jnp.int32) * NPAD)[:, None])
    idxw = idxw.reshape(K, NWIN, WR).transpose(1, 0, 2)

    s0, ss0, eo = _stats1(feats, emb8, We, bevec)
    a1, b1 = _affine(s0, ss0, gamma1, beta1)
    y1 = _mm27(feats, a1, b1, W1, b1vec)
    h1s = _sc_gather_sum(y1, idxw, None)

    s1, ss1 = _stats2(h1s, eo)
    a2, b2 = _affine(s1, ss1, gamma2, beta2)
    y2 = _mm27(h1s, a2, b2, W2, b2vec, eo=eo)
    out = _sc_gather_sum(y2, idxw, feats)
    return out


# speedup vs baseline: 14.5116x; 1.1612x over previous
import functools

import jax
import jax.numpy as jnp
from jax import lax
from jax.experimental import pallas as pl
from jax.experimental.pallas import tpu as pltpu
from jax.experimental.pallas import tpu_sc as plsc

N = 200000
B = 4
C = 64
G = 32
K = 27
NPER = 50000
T = 1000
NT = N // T
TPB = NPER // T
TM = 2000
NTM = N // TM
TPBM = NPER // TM
NPAD = N + TM
WR = 400
NWIN = N // WR
NWORK = 32
SENT = N

_f32 = jnp.float32


def _row4(ref, bidx):
    r = ref[0:1, :]
    for b in range(1, B):
        r = jnp.where(bidx == b, ref[b:b + 1, :], r)
    return r


def _stats1_body(x_ref, emb_ref, we_ref, bevec_ref, s_ref, ss_ref, eo_ref):
    t = pl.program_id(0)
    bidx = t // TPB
    x = x_ref[...]

    @pl.when(t == 0)
    def _():
        s_ref[...] = jnp.zeros_like(s_ref)
        ss_ref[...] = jnp.zeros_like(ss_ref)
        e = emb_ref[...]
        he = e * jax.nn.sigmoid(e)
        eo_ref[...] = (
            lax.dot_general(he, we_ref[...], (((1,), (0,)), ((), ())),
                            precision=lax.Precision.HIGHEST,
                            preferred_element_type=_f32)
            + bevec_ref[...]
        )

    oh = (lax.broadcasted_iota(jnp.int32, (8, 1), 0) == bidx).astype(_f32)
    s_ref[...] += oh * jnp.sum(x, axis=0)[None, :]
    ss_ref[...] += oh * jnp.sum(x * x, axis=0)[None, :]


def _stats2_body(x_ref, eo_ref, s_ref, ss_ref):
    t = pl.program_id(0)
    bidx = t // TPB
    x = x_ref[...] + _row4(eo_ref, bidx)

    @pl.when(t == 0)
    def _():
        s_ref[...] = jnp.zeros_like(s_ref)
        ss_ref[...] = jnp.zeros_like(ss_ref)

    oh = (lax.broadcasted_iota(jnp.int32, (8, 1), 0) == bidx).astype(_f32)
    s_ref[...] += oh * jnp.sum(x, axis=0)[None, :]
    ss_ref[...] += oh * jnp.sum(x * x, axis=0)[None, :]


def _mm_body(x_ref, a_ref, b_ref, w_ref, cb_ref, y_ref, *, add_emb, eo_ref=None):
    t = pl.program_id(0)
    bidx = jnp.minimum(t, NTM - 1) // TPBM
    x = x_ref[...]
    if add_emb:
        x = x + _row4(eo_ref, bidx)
    hp = x * _row4(a_ref, bidx) + _row4(b_ref, bidx)
    h = hp * jax.nn.sigmoid(hp)
    live = (t < NTM).astype(_f32)
    h = h * live
    y_ref[...] = (
        lax.dot_general(h, w_ref[...], (((1,), (0,)), ((), ())),
                        preferred_element_type=_f32)
        + cb_ref[0:1, :] * live
    )


def _mm_body_noemb(x_ref, a_ref, b_ref, w_ref, cb_ref, y_ref):
    _mm_body(x_ref, a_ref, b_ref, w_ref, cb_ref, y_ref, add_emb=False)


def _mm_body_emb(x_ref, eo_ref, a_ref, b_ref, w_ref, cb_ref, y_ref):
    _mm_body(x_ref, a_ref, b_ref, w_ref, cb_ref, y_ref, add_emb=True,
             eo_ref=eo_ref)


_VEC8 = pl.BlockSpec((8, C), lambda t: (0, 0))


def _stats1(feats, emb8, We, bevec):
    return pl.pallas_call(
        _stats1_body,
        grid=(NT,),
        in_specs=[
            pl.BlockSpec((T, C), lambda t: (t, 0)),
            pl.BlockSpec((8, 512), lambda t: (0, 0)),
            pl.BlockSpec((512, C), lambda t: (0, 0)),
            _VEC8,
        ],
        out_specs=[_VEC8, _VEC8, _VEC8],
        out_shape=[jax.ShapeDtypeStruct((8, C), _f32)] * 3,
    )(feats, emb8, We, bevec)


def _stats2(h1s, eo):
    return pl.pallas_call(
        _stats2_body,
        grid=(NT,),
        in_specs=[pl.BlockSpec((T, C), lambda t: (t, 0)), _VEC8],
        out_specs=[_VEC8, _VEC8],
        out_shape=[jax.ShapeDtypeStruct((8, C), _f32)] * 2,
    )(h1s, eo)


def _mm27(x, a, b, Wcat, cbvec, eo=None):
    body = _mm_body_noemb if eo is None else _mm_body_emb
    xs = [x] if eo is None else [x, eo]
    in_specs = [pl.BlockSpec((TM, C), lambda t: (jnp.minimum(t, NTM - 1), 0))]
    if eo is not None:
        in_specs.append(_VEC8)
    in_specs += [
        _VEC8, _VEC8,
        pl.BlockSpec((C, K * C), lambda t: (0, 0)),
        pl.BlockSpec((8, K * C), lambda t: (0, 0)),
    ]
    y = pl.pallas_call(
        body,
        grid=(NTM + 1,),
        in_specs=in_specs,
        out_specs=pl.BlockSpec((TM, K * C), lambda t: (t, 0)),
        out_shape=jax.ShapeDtypeStruct((NPAD, K * C), _f32),
    )(*xs, a, b, Wcat, cbvec)
    return y.reshape(NPAD * K, C)


def _sc_gather_sum(yflat, idxw, resid):
    mesh = plsc.VectorSubcoreMesh(core_axis_name="c", subcore_axis_name="s")
    with_resid = resid is not None

    @functools.partial(
        pl.kernel,
        out_type=jax.ShapeDtypeStruct((N, C), _f32),
        mesh=mesh,
        compiler_params=pltpu.CompilerParams(use_tc_tiling_on_sc=False),
        scratch_types=[
            pltpu.VMEM((K, WR), jnp.int32),
            pltpu.VMEM((WR,), jnp.int32),
            pltpu.VMEM((WR, C), _f32),
            pltpu.VMEM((WR, C), _f32),
            pltpu.VMEM((WR, C), _f32),
            pltpu.VMEM_SHARED((16 * WR, C), _f32),
            pltpu.SemaphoreType.DMA,
            pltpu.SemaphoreType.DMA,
            pltpu.SemaphoreType.DMA,
        ],
    )
    def k(*refs):
        if with_resid:
            y_hbm, i_hbm, r_hbm, o_hbm = refs[:4]
        else:
            y_hbm, i_hbm, o_hbm = refs[:3]
        idxw_v, idv_v, z_v, ga_v, gb_v, acc_sh, sem_a, sem_b, sem_m = refs[-9:]
        cid = lax.axis_index("c")
        sid = lax.axis_index("s")
        wid = sid * 2 + cid

        @pl.loop(0, WR // 16)
        def _(j):
            sl = pl.ds(j * 16, 16)
            idv_v[sl] = jnp.arange(16, dtype=jnp.int32) + (j * 16 + sid * WR)

        @pl.loop(0, WR)
        def _(r):
            for cc in range(C // 16):
                z_v[r, pl.ds(cc * 16, 16)] = jnp.zeros((16,), _f32)

        def gather(kk, buf, sem):
            return pltpu.async_copy(y_hbm.at[idxw_v.at[kk]], buf, sem)

        def scat_add(buf):
            pltpu.sync_copy(buf, acc_sh.at[idv_v], add=True)

        @pl.loop(0, NWIN // NWORK + 1)
        def _(it):
            w = wid + NWORK * it

            @pl.when(w < NWIN)
            def _():
                base = w * WR
                pltpu.async_copy(i_hbm.at[w], idxw_v, sem_m).wait()
                if with_resid:
                    pltpu.async_copy(
                        r_hbm.at[pl.ds(base, WR)],
                        acc_sh.at[pl.ds(sid * WR, WR)], sem_m).wait()
                else:
                    pltpu.async_copy(
                        z_v, acc_sh.at[pl.ds(sid * WR, WR)], sem_m).wait()
                gather(0, ga_v, sem_a)

                @pl.loop(0, (K - 1) // 2)
                def _(j):
                    gather(2 * j + 1, gb_v, sem_b)
                    pltpu.make_async_copy(y_hbm.at[idxw_v.at[0]], ga_v,
                                          sem_a).wait()
                    scat_add(ga_v)
                    gather(2 * j + 2, ga_v, sem_a)
                    pltpu.make_async_copy(y_hbm.at[idxw_v.at[0]], gb_v,
                                          sem_b).wait()
                    scat_add(gb_v)

                pltpu.make_async_copy(y_hbm.at[idxw_v.at[0]], ga_v,
                                      sem_a).wait()
                scat_add(ga_v)
                pltpu.async_copy(acc_sh.at[pl.ds(sid * WR, WR)],
                                 o_hbm.at[pl.ds(base, WR)], sem_m).wait()

    args = (yflat, idxw, resid) if with_resid else (yflat, idxw)
    return k(*args)


def _affine(s, ss, gamma, beta):
    cnt = float(NPER * (C // G))
    sg = s[:B].reshape(B, G, C // G).sum(-1)
    ssg = ss[:B].reshape(B, G, C // G).sum(-1)
    mean = sg / cnt
    var = ssg / cnt - mean * mean
    rstd = lax.rsqrt(var + 1e-5)
    meanc = jnp.repeat(mean, C // G, axis=-1)
    rstdc = jnp.repeat(rstd, C // G, axis=-1)
    a = rstdc * gamma[None, :]
    b = beta[None, :] - meanc * a
    pad = jnp.zeros((8 - B, C), _f32)
    return jnp.concatenate([a, pad], 0), jnp.concatenate([b, pad], 0)


def kernel(feats, emb, gamma1, beta1, W1, b1c, We, be, gamma2, beta2, W2, b2c,
           batch_idx, nbrs):
    emb8 = jnp.concatenate([emb, jnp.zeros((8 - B, emb.shape[1]), _f32)], 0)
    bevec = jnp.broadcast_to(be[None, :], (8, C))
    z8 = jnp.zeros((8, C), _f32)
    b1vec = jnp.concatenate(
        [z8[:, :0]] + [jnp.broadcast_to(b1c[None, :], (8, C)) if k == 13 else z8
                       for k in range(K)], axis=1)
    b2vec = jnp.concatenate(
        [jnp.broadcast_to(b2c[None, :], (8, C)) if k == 13 else z8
         for k in range(K)], axis=1)
    W1cat = W1.transpose(1, 0, 2).reshape(C, K * C)
    W2cat = W2.transpose(1, 0, 2).reshape(C, K * C)
    idxw = (jnp.minimum(nbrs, SENT) * K
            + jnp.arange(K, dtype=jnp.int32)[:, None])
    idxw = idxw.reshape(K, NWIN, WR).transpose(1, 0, 2)

    s0, ss0, eo = _stats1(feats, emb8, We, bevec)
    a1, b1 = _affine(s0, ss0, gamma1, beta1)
    y1 = _mm27(feats, a1, b1, W1cat, b1vec)
    h1s = _sc_gather_sum(y1, idxw, None)

    s1, ss1 = _stats2(h1s, eo)
    a2, b2 = _affine(s1, ss1, gamma2, beta2)
    y2 = _mm27(h1s, a2, b2, W2cat, b2vec, eo=eo)
    out = _sc_gather_sum(y2, idxw, feats)
    return out


# speedup vs baseline: 54.2211x; 3.7364x over previous
import functools

import jax
import jax.numpy as jnp
from jax import lax
from jax.experimental import pallas as pl
from jax.experimental.pallas import tpu as pltpu
from jax.experimental.pallas import tpu_sc as plsc

N = 200000
B = 4
C = 64
G = 32
K = 27
NPER = 50000
T = 1000
NT = N // T
TPB = NPER // T
TM = 2000
NTM = N // TM
TPBM = NPER // TM
NPAD = N + TM
WR = 400
NWIN = N // WR
NWORK = 32
SENT = N

_f32 = jnp.float32


def _row4(ref, bidx):
    r = ref[0:1, :]
    for b in range(1, B):
        r = jnp.where(bidx == b, ref[b:b + 1, :], r)
    return r


def _stats1_body(x_ref, emb_ref, we_ref, bevec_ref, s_ref, ss_ref, eo_ref):
    t = pl.program_id(0)
    bidx = t // TPB
    x = x_ref[...]

    @pl.when(t == 0)
    def _():
        s_ref[...] = jnp.zeros_like(s_ref)
        ss_ref[...] = jnp.zeros_like(ss_ref)
        e = emb_ref[...]
        he = e * jax.nn.sigmoid(e)
        eo_ref[...] = (
            lax.dot_general(he, we_ref[...], (((1,), (0,)), ((), ())),
                            precision=lax.Precision.HIGHEST,
                            preferred_element_type=_f32)
            + bevec_ref[...]
        )

    oh = (lax.broadcasted_iota(jnp.int32, (8, 1), 0) == bidx).astype(_f32)
    s_ref[...] += oh * jnp.sum(x, axis=0)[None, :]
    ss_ref[...] += oh * jnp.sum(x * x, axis=0)[None, :]


def _stats2_body(x_ref, eo_ref, s_ref, ss_ref):
    t = pl.program_id(0)
    bidx = t // TPB
    x = x_ref[...] + _row4(eo_ref, bidx)

    @pl.when(t == 0)
    def _():
        s_ref[...] = jnp.zeros_like(s_ref)
        ss_ref[...] = jnp.zeros_like(ss_ref)

    oh = (lax.broadcasted_iota(jnp.int32, (8, 1), 0) == bidx).astype(_f32)
    s_ref[...] += oh * jnp.sum(x, axis=0)[None, :]
    ss_ref[...] += oh * jnp.sum(x * x, axis=0)[None, :]


def _mm_body(x_ref, a_ref, b_ref, w_ref, cb_ref, y_ref, *, add_emb, eo_ref=None):
    t = pl.program_id(0)
    bidx = jnp.minimum(t, NTM - 1) // TPBM
    x = x_ref[...]
    if add_emb:
        x = x + _row4(eo_ref, bidx)
    hp = x * _row4(a_ref, bidx) + _row4(b_ref, bidx)
    h = hp * jax.nn.sigmoid(hp)
    live = (t < NTM).astype(_f32)
    h = h * live
    y_ref[...] = (
        lax.dot_general(h, w_ref[...], (((1,), (0,)), ((), ())),
                        preferred_element_type=_f32)
        + cb_ref[0:1, :] * live
    )


def _mm_body_noemb(x_ref, a_ref, b_ref, w_ref, cb_ref, y_ref):
    _mm_body(x_ref, a_ref, b_ref, w_ref, cb_ref, y_ref, add_emb=False)


def _mm_body_emb(x_ref, eo_ref, a_ref, b_ref, w_ref, cb_ref, y_ref):
    _mm_body(x_ref, a_ref, b_ref, w_ref, cb_ref, y_ref, add_emb=True,
             eo_ref=eo_ref)


_VEC8 = pl.BlockSpec((8, C), lambda t: (0, 0))


def _stats1(feats, emb8, We, bevec):
    return pl.pallas_call(
        _stats1_body,
        grid=(NT,),
        in_specs=[
            pl.BlockSpec((T, C), lambda t: (t, 0)),
            pl.BlockSpec((8, 512), lambda t: (0, 0)),
            pl.BlockSpec((512, C), lambda t: (0, 0)),
            _VEC8,
        ],
        out_specs=[_VEC8, _VEC8, _VEC8],
        out_shape=[jax.ShapeDtypeStruct((8, C), _f32)] * 3,
    )(feats, emb8, We, bevec)


def _stats2(h1s, eo):
    return pl.pallas_call(
        _stats2_body,
        grid=(NT,),
        in_specs=[pl.BlockSpec((T, C), lambda t: (t, 0)), _VEC8],
        out_specs=[_VEC8, _VEC8],
        out_shape=[jax.ShapeDtypeStruct((8, C), _f32)] * 2,
    )(h1s, eo)


def _mm27(x, a, b, Wcat, cbvec, eo=None):
    body = _mm_body_noemb if eo is None else _mm_body_emb
    xs = [x] if eo is None else [x, eo]
    in_specs = [pl.BlockSpec((TM, C), lambda t: (jnp.minimum(t, NTM - 1), 0))]
    if eo is not None:
        in_specs.append(_VEC8)
    in_specs += [
        _VEC8, _VEC8,
        pl.BlockSpec((C, K * C), lambda t: (0, 0)),
        pl.BlockSpec((8, K * C), lambda t: (0, 0)),
    ]
    y = pl.pallas_call(
        body,
        grid=(NTM + 1,),
        in_specs=in_specs,
        out_specs=pl.BlockSpec((TM, K * C), lambda t: (t, 0)),
        out_shape=jax.ShapeDtypeStruct((NPAD, K * C), _f32),
    )(*xs, a, b, Wcat, cbvec)
    return y.reshape(NPAD * K, C)


def _sc_gather_sum(yflat, idxw, resid):
    mesh = plsc.VectorSubcoreMesh(core_axis_name="c", subcore_axis_name="s")
    with_resid = resid is not None

    @functools.partial(
        pl.kernel,
        out_type=jax.ShapeDtypeStruct((N, C), _f32),
        mesh=mesh,
        compiler_params=pltpu.CompilerParams(use_tc_tiling_on_sc=False),
        scratch_types=[
            pltpu.VMEM((K, WR), jnp.int32),
            pltpu.VMEM((WR,), jnp.int32),
            pltpu.VMEM((WR, C), _f32),
            pltpu.VMEM((WR, C), _f32),
            pltpu.VMEM((WR, C), _f32),
            pltpu.VMEM_SHARED((16 * WR, C), _f32),
            pltpu.SemaphoreType.DMA,
            pltpu.SemaphoreType.DMA,
            pltpu.SemaphoreType.DMA,
        ],
    )
    def k(*refs):
        if with_resid:
            y_hbm, i_hbm, r_hbm, o_hbm = refs[:4]
        else:
            y_hbm, i_hbm, o_hbm = refs[:3]
        idxw_v, idv_v, z_v, ga_v, gb_v, acc_sh, sem_a, sem_b, sem_m = refs[-9:]
        cid = lax.axis_index("c")
        sid = lax.axis_index("s")
        wid = sid * 2 + cid

        @pl.loop(0, WR // 16)
        def _(j):
            sl = pl.ds(j * 16, 16)
            idv_v[sl] = jnp.arange(16, dtype=jnp.int32) + (j * 16 + sid * WR)

        @pl.loop(0, WR)
        def _(r):
            for cc in range(C // 16):
                z_v[r, pl.ds(cc * 16, 16)] = jnp.zeros((16,), _f32)

        def gather(kk, buf, sem):
            return pltpu.async_copy(y_hbm.at[idxw_v.at[kk]], buf, sem)

        def scat_add(buf):
            pltpu.sync_copy(buf, acc_sh.at[idv_v], add=True)

        @pl.loop(0, NWIN // NWORK + 1)
        def _(it):
            w = wid + NWORK * it

            @pl.when(w < NWIN)
            def _():
                base = w * WR
                pltpu.async_copy(i_hbm.at[w], idxw_v, sem_m).wait()
                if with_resid:
                    pltpu.async_copy(
                        r_hbm.at[pl.ds(base, WR)],
                        acc_sh.at[pl.ds(sid * WR, WR)], sem_m).wait()
                else:
                    pltpu.async_copy(
                        z_v, acc_sh.at[pl.ds(sid * WR, WR)], sem_m).wait()
                gather(0, ga_v, sem_a)

                @pl.loop(0, (K - 1) // 2)
                def _(j):
                    gather(2 * j + 1, gb_v, sem_b)
                    pltpu.make_async_copy(y_hbm.at[idxw_v.at[0]], ga_v,
                                          sem_a).wait()
                    scat_add(ga_v)
                    gather(2 * j + 2, ga_v, sem_a)
                    pltpu.make_async_copy(y_hbm.at[idxw_v.at[0]], gb_v,
                                          sem_b).wait()
                    scat_add(gb_v)

                pltpu.make_async_copy(y_hbm.at[idxw_v.at[0]], ga_v,
                                      sem_a).wait()
                scat_add(ga_v)
                pltpu.async_copy(acc_sh.at[pl.ds(sid * WR, WR)],
                                 o_hbm.at[pl.ds(base, WR)], sem_m).wait()

    args = (yflat, idxw, resid) if with_resid else (yflat, idxw)
    return k(*args)


def _affine(s, ss, gamma, beta):
    cnt = float(NPER * (C // G))
    sg = s[:B].reshape(B, G, C // G).sum(-1)
    ssg = ss[:B].reshape(B, G, C // G).sum(-1)
    mean = sg / cnt
    var = ssg / cnt - mean * mean
    rstd = lax.rsqrt(var + 1e-5)
    meanc = jnp.repeat(mean, C // G, axis=-1)
    rstdc = jnp.repeat(rstd, C // G, axis=-1)
    a = rstdc * gamma[None, :]
    b = beta[None, :] - meanc * a
    pad = jnp.zeros((8 - B, C), _f32)
    return jnp.concatenate([a, pad], 0), jnp.concatenate([b, pad], 0)


def kernel(feats, emb, gamma1, beta1, W1, b1c, We, be, gamma2, beta2, W2, b2c,
           batch_idx, nbrs):
    emb8 = jnp.concatenate([emb, jnp.zeros((8 - B, emb.shape[1]), _f32)], 0)
    bevec = jnp.broadcast_to(be[None, :], (8, C))
    z8 = jnp.zeros((8, C), _f32)
    b1vec = jnp.concatenate(
        [z8[:, :0]] + [jnp.broadcast_to(b1c[None, :], (8, C)) if k == 13 else z8
                       for k in range(K)], axis=1)
    b2vec = jnp.concatenate(
        [jnp.broadcast_to(b2c[None, :], (8, C)) if k == 13 else z8
         for k in range(K)], axis=1)
    W1cat = W1.transpose(1, 0, 2).reshape(C, K * C)
    W2cat = W2.transpose(1, 0, 2).reshape(C, K * C)
    spread = SENT + (jnp.arange(N, dtype=jnp.int32) % TM)[None, :]
    idxw = (jnp.where(nbrs < SENT, nbrs, spread) * K
            + jnp.arange(K, dtype=jnp.int32)[:, None])
    idxw = idxw.reshape(K, NWIN, WR).transpose(1, 0, 2)

    s0, ss0, eo = _stats1(feats, emb8, We, bevec)
    a1, b1 = _affine(s0, ss0, gamma1, beta1)
    y1 = _mm27(feats, a1, b1, W1cat, b1vec)
    h1s = _sc_gather_sum(y1, idxw, None)

    s1, ss1 = _stats2(h1s, eo)
    a2, b2 = _affine(s1, ss1, gamma2, beta2)
    y2 = _mm27(h1s, a2, b2, W2cat, b2vec, eo=eo)
    out = _sc_gather_sum(y2, idxw, feats)
    return out


# speedup vs baseline: 54.6146x; 1.0073x over previous
import functools

import jax
import jax.numpy as jnp
from jax import lax
from jax.experimental import pallas as pl
from jax.experimental.pallas import tpu as pltpu
from jax.experimental.pallas import tpu_sc as plsc

N = 200000
B = 4
C = 64
G = 32
K = 27
NPER = 50000
T = 1000
NT = N // T
TPB = NPER // T
TM = 2000
NTM = N // TM
TPBM = NPER // TM
NPAD = N + TM
WR = 400
NWIN = N // WR
NWORK = 32
SENT = N

_f32 = jnp.float32


def _row4(ref, bidx):
    r = ref[0:1, :]
    for b in range(1, B):
        r = jnp.where(bidx == b, ref[b:b + 1, :], r)
    return r


def _stats1_body(x_ref, emb_ref, we_ref, bevec_ref, s_ref, ss_ref, eo_ref):
    t = pl.program_id(0)
    bidx = t // TPB
    x = x_ref[...]

    @pl.when(t == 0)
    def _():
        s_ref[...] = jnp.zeros_like(s_ref)
        ss_ref[...] = jnp.zeros_like(ss_ref)
        e = emb_ref[...]
        he = e * jax.nn.sigmoid(e)
        eo_ref[...] = (
            lax.dot_general(he, we_ref[...], (((1,), (0,)), ((), ())),
                            precision=lax.Precision.HIGHEST,
                            preferred_element_type=_f32)
            + bevec_ref[...]
        )

    oh = (lax.broadcasted_iota(jnp.int32, (8, 1), 0) == bidx).astype(_f32)
    s_ref[...] += oh * jnp.sum(x, axis=0)[None, :]
    ss_ref[...] += oh * jnp.sum(x * x, axis=0)[None, :]


def _stats2_body(x_ref, eo_ref, s_ref, ss_ref):
    t = pl.program_id(0)
    bidx = t // TPB
    x = x_ref[...] + _row4(eo_ref, bidx)

    @pl.when(t == 0)
    def _():
        s_ref[...] = jnp.zeros_like(s_ref)
        ss_ref[...] = jnp.zeros_like(ss_ref)

    oh = (lax.broadcasted_iota(jnp.int32, (8, 1), 0) == bidx).astype(_f32)
    s_ref[...] += oh * jnp.sum(x, axis=0)[None, :]
    ss_ref[...] += oh * jnp.sum(x * x, axis=0)[None, :]


def _mm_body(x_ref, a_ref, b_ref, w_ref, cb_ref, y_ref, *, add_emb, eo_ref=None):
    t = pl.program_id(0)
    bidx = jnp.minimum(t, NTM - 1) // TPBM
    x = x_ref[...]
    if add_emb:
        x = x + _row4(eo_ref, bidx)
    hp = x * _row4(a_ref, bidx) + _row4(b_ref, bidx)
    h = hp * jax.nn.sigmoid(hp)
    live = (t < NTM).astype(_f32)
    h = h * live
    y_ref[...] = (
        lax.dot_general(h, w_ref[...], (((1,), (0,)), ((), ())),
                        preferred_element_type=_f32)
        + cb_ref[0:1, :] * live
    )


def _mm_body_noemb(x_ref, a_ref, b_ref, w_ref, cb_ref, y_ref):
    _mm_body(x_ref, a_ref, b_ref, w_ref, cb_ref, y_ref, add_emb=False)


def _mm_body_emb(x_ref, eo_ref, a_ref, b_ref, w_ref, cb_ref, y_ref):
    _mm_body(x_ref, a_ref, b_ref, w_ref, cb_ref, y_ref, add_emb=True,
             eo_ref=eo_ref)


_VEC8 = pl.BlockSpec((8, C), lambda t: (0, 0))


def _stats1(feats, emb8, We, bevec):
    return pl.pallas_call(
        _stats1_body,
        grid=(NT,),
        in_specs=[
            pl.BlockSpec((T, C), lambda t: (t, 0)),
            pl.BlockSpec((8, 512), lambda t: (0, 0)),
            pl.BlockSpec((512, C), lambda t: (0, 0)),
            _VEC8,
        ],
        out_specs=[_VEC8, _VEC8, _VEC8],
        out_shape=[jax.ShapeDtypeStruct((8, C), _f32)] * 3,
    )(feats, emb8, We, bevec)


def _stats2(h1s, eo):
    return pl.pallas_call(
        _stats2_body,
        grid=(NT,),
        in_specs=[pl.BlockSpec((T, C), lambda t: (t, 0)), _VEC8],
        out_specs=[_VEC8, _VEC8],
        out_shape=[jax.ShapeDtypeStruct((8, C), _f32)] * 2,
    )(h1s, eo)


def _mm27(x, a, b, Wcat, cbvec, eo=None):
    body = _mm_body_noemb if eo is None else _mm_body_emb
    xs = [x] if eo is None else [x, eo]
    in_specs = [pl.BlockSpec((TM, C), lambda t: (jnp.minimum(t, NTM - 1), 0))]
    if eo is not None:
        in_specs.append(_VEC8)
    in_specs += [
        _VEC8, _VEC8,
        pl.BlockSpec((C, K * C), lambda t: (0, 0)),
        pl.BlockSpec((8, K * C), lambda t: (0, 0)),
    ]
    y = pl.pallas_call(
        body,
        grid=(NTM + 1,),
        in_specs=in_specs,
        out_specs=pl.BlockSpec((TM, K * C), lambda t: (t, 0)),
        out_shape=jax.ShapeDtypeStruct((NPAD, K * C), _f32),
    )(*xs, a, b, Wcat, cbvec)
    return y.reshape(NPAD * K, C)


def _sc_gather_sum(yflat, idxw, resid):
    mesh = plsc.VectorSubcoreMesh(core_axis_name="c", subcore_axis_name="s")
    with_resid = resid is not None

    @functools.partial(
        pl.kernel,
        out_type=jax.ShapeDtypeStruct((N, C), _f32),
        mesh=mesh,
        compiler_params=pltpu.CompilerParams(use_tc_tiling_on_sc=False),
        scratch_types=[
            pltpu.VMEM((K, WR), jnp.int32),
            pltpu.VMEM((WR,), jnp.int32),
            pltpu.VMEM((WR, C), _f32),
            pltpu.VMEM((WR, C), _f32),
            pltpu.VMEM((WR, C), _f32),
            pltpu.VMEM_SHARED((16 * WR, C), _f32),
            pltpu.SemaphoreType.DMA,
            pltpu.SemaphoreType.DMA,
            pltpu.SemaphoreType.DMA,
        ],
    )
    def k(*refs):
        if with_resid:
            y_hbm, i_hbm, r_hbm, o_hbm = refs[:4]
        else:
            y_hbm, i_hbm, o_hbm = refs[:3]
        idxw_v, idv_v, z_v, ga_v, gb_v, acc_sh, sem_a, sem_b, sem_m = refs[-9:]
        cid = lax.axis_index("c")
        sid = lax.axis_index("s")
        wid = sid * 2 + cid

        @pl.loop(0, WR // 16)
        def _(j):
            sl = pl.ds(j * 16, 16)
            idv_v[sl] = jnp.arange(16, dtype=jnp.int32) + (j * 16 + sid * WR)

        @pl.loop(0, WR)
        def _(r):
            for cc in range(C // 16):
                z_v[r, pl.ds(cc * 16, 16)] = jnp.zeros((16,), _f32)

        def gather(kk, buf, sem):
            return pltpu.async_copy(y_hbm.at[idxw_v.at[kk]], buf, sem)

        def scat_add(buf):
            pltpu.sync_copy(buf, acc_sh.at[idv_v], add=True)

        @pl.loop(0, NWIN // NWORK + 1)
        def _(it):
            w = wid + NWORK * it

            @pl.when(w < NWIN)
            def _():
                base = w * WR
                pltpu.async_copy(i_hbm.at[:, w, :], idxw_v, sem_m).wait()
                if with_resid:
                    pltpu.async_copy(
                        r_hbm.at[pl.ds(base, WR)],
                        acc_sh.at[pl.ds(sid * WR, WR)], sem_m).wait()
                else:
                    pltpu.async_copy(
                        z_v, acc_sh.at[pl.ds(sid * WR, WR)], sem_m).wait()
                gather(0, ga_v, sem_a)

                @pl.loop(0, (K - 1) // 2)
                def _(j):
                    gather(2 * j + 1, gb_v, sem_b)
                    pltpu.make_async_copy(y_hbm.at[idxw_v.at[0]], ga_v,
                                          sem_a).wait()
                    scat_add(ga_v)
                    gather(2 * j + 2, ga_v, sem_a)
                    pltpu.make_async_copy(y_hbm.at[idxw_v.at[0]], gb_v,
                                          sem_b).wait()
                    scat_add(gb_v)

                pltpu.make_async_copy(y_hbm.at[idxw_v.at[0]], ga_v,
                                      sem_a).wait()
                scat_add(ga_v)
                pltpu.async_copy(acc_sh.at[pl.ds(sid * WR, WR)],
                                 o_hbm.at[pl.ds(base, WR)], sem_m).wait()

    args = (yflat, idxw, resid) if with_resid else (yflat, idxw)
    return k(*args)


def _affine(s, ss, gamma, beta):
    cnt = float(NPER * (C // G))
    sg = s[:B].reshape(B, G, C // G).sum(-1)
    ssg = ss[:B].reshape(B, G, C // G).sum(-1)
    mean = sg / cnt
    var = ssg / cnt - mean * mean
    rstd = lax.rsqrt(var + 1e-5)
    meanc = jnp.repeat(mean, C // G, axis=-1)
    rstdc = jnp.repeat(rstd, C // G, axis=-1)
    a = rstdc * gamma[None, :]
    b = beta[None, :] - meanc * a
    pad = jnp.zeros((8 - B, C), _f32)
    return jnp.concatenate([a, pad], 0), jnp.concatenate([b, pad], 0)


def kernel(feats, emb, gamma1, beta1, W1, b1c, We, be, gamma2, beta2, W2, b2c,
           batch_idx, nbrs):
    emb8 = jnp.concatenate([emb, jnp.zeros((8 - B, emb.shape[1]), _f32)], 0)
    bevec = jnp.broadcast_to(be[None, :], (8, C))
    z8 = jnp.zeros((8, C), _f32)
    b1vec = jnp.concatenate(
        [z8[:, :0]] + [jnp.broadcast_to(b1c[None, :], (8, C)) if k == 13 else z8
                       for k in range(K)], axis=1)
    b2vec = jnp.concatenate(
        [jnp.broadcast_to(b2c[None, :], (8, C)) if k == 13 else z8
         for k in range(K)], axis=1)
    W1cat = W1.transpose(1, 0, 2).reshape(C, K * C)
    W2cat = W2.transpose(1, 0, 2).reshape(C, K * C)
    spread = SENT + (jnp.arange(N, dtype=jnp.int32) % TM)[None, :]
    idxw = (jnp.where(nbrs < SENT, nbrs, spread) * K
            + jnp.arange(K, dtype=jnp.int32)[:, None])
    idxw = idxw.reshape(K, NWIN, WR)

    s0, ss0, eo = _stats1(feats, emb8, We, bevec)
    a1, b1 = _affine(s0, ss0, gamma1, beta1)
    y1 = _mm27(feats, a1, b1, W1cat, b1vec)
    h1s = _sc_gather_sum(y1, idxw, None)

    s1, ss1 = _stats2(h1s, eo)
    a2, b2 = _affine(s1, ss1, gamma2, beta2)
    y2 = _mm27(h1s, a2, b2, W2cat, b2vec, eo=eo)
    out = _sc_gather_sum(y2, idxw, feats)
    return out


# speedup vs baseline: 57.2364x; 1.0480x over previous
import functools

import jax
import jax.numpy as jnp
from jax import lax
from jax.experimental import pallas as pl
from jax.experimental.pallas import tpu as pltpu
from jax.experimental.pallas import tpu_sc as plsc

N = 200000
B = 4
C = 64
G = 32
K = 27
NB = 50000
T = 1000
NTB = NB // T
TM = 2000
NTM = NB // TM
NPADB = NB + TM
WR = 400
NWINB = NB // WR
NWORK = 32
SENT = N

_f32 = jnp.float32
_VEC8 = pl.BlockSpec((8, C), lambda t: (0, 0))


def _emb_body(emb_ref, we_ref, bevec_ref, eo_ref):
    e = emb_ref[...]
    he = e * jax.nn.sigmoid(e)
    eo_ref[...] = (
        lax.dot_general(he, we_ref[...], (((1,), (0,)), ((), ())),
                        preferred_element_type=_f32)
        + bevec_ref[...]
    )


def _emb_mlp(emb8, We, bevec):
    return pl.pallas_call(
        _emb_body,
        grid=(1,),
        in_specs=[
            pl.BlockSpec((8, 512), lambda t: (0, 0)),
            pl.BlockSpec((512, C), lambda t: (0, 0)),
            _VEC8,
        ],
        out_specs=_VEC8,
        out_shape=jax.ShapeDtypeStruct((8, C), _f32),
    )(emb8, We, bevec)


def _stats_body(x_ref, eo_ref, s_ref, ss_ref):
    t = pl.program_id(0)
    x = x_ref[...] + eo_ref[0:1, :]

    @pl.when(t == 0)
    def _():
        s_ref[...] = jnp.zeros_like(s_ref)
        ss_ref[...] = jnp.zeros_like(ss_ref)

    s_ref[0:1, :] += jnp.sum(x, axis=0)[None, :]
    ss_ref[0:1, :] += jnp.sum(x * x, axis=0)[None, :]


def _stats(xb, eob):
    return pl.pallas_call(
        _stats_body,
        grid=(NTB,),
        in_specs=[pl.BlockSpec((T, C), lambda t: (t, 0)), _VEC8],
        out_specs=[_VEC8, _VEC8],
        out_shape=[jax.ShapeDtypeStruct((8, C), _f32)] * 2,
    )(xb, eob)


def _mm_body(x_ref, eo_ref, a_ref, b_ref, w_ref, cb_ref, y_ref):
    t = pl.program_id(0)
    x = x_ref[...] + eo_ref[0:1, :]
    hp = x * a_ref[0:1, :] + b_ref[0:1, :]
    h = hp * jax.nn.sigmoid(hp)
    live = (t < NTM).astype(_f32)
    h = h * live
    y_ref[...] = (
        lax.dot_general(h, w_ref[...], (((1,), (0,)), ((), ())),
                        preferred_element_type=_f32)
        + cb_ref[0:1, :] * live
    )


def _mm27(xb, eob, a, b, Wcat, cbvec):
    y = pl.pallas_call(
        _mm_body,
        grid=(NTM + 1,),
        in_specs=[
            pl.BlockSpec((TM, C), lambda t: (jnp.minimum(t, NTM - 1), 0)),
            _VEC8, _VEC8, _VEC8,
            pl.BlockSpec((C, K * C), lambda t: (0, 0)),
            pl.BlockSpec((8, K * C), lambda t: (0, 0)),
        ],
        out_specs=pl.BlockSpec((TM, K * C), lambda t: (t, 0)),
        out_shape=jax.ShapeDtypeStruct((NPADB, K * C), _f32),
    )(xb, eob, a, b, Wcat, cbvec)
    return y.reshape(NPADB * K, C)


def _sc_gather_sum(yflat, idxw, resid):
    mesh = plsc.VectorSubcoreMesh(core_axis_name="c", subcore_axis_name="s")
    with_resid = resid is not None
    _, nwin, wr = idxw.shape
    n = nwin * wr
    iters = (nwin + NWORK - 1) // NWORK

    @functools.partial(
        pl.kernel,
        out_type=jax.ShapeDtypeStruct((n, C), _f32),
        mesh=mesh,
        compiler_params=pltpu.CompilerParams(use_tc_tiling_on_sc=False),
        scratch_types=[
            pltpu.VMEM((K, wr), jnp.int32),
            pltpu.VMEM((wr,), jnp.int32),
            pltpu.VMEM((wr, C), _f32),
            pltpu.VMEM((wr, C), _f32),
            pltpu.VMEM((wr, C), _f32),
            pltpu.VMEM_SHARED((16 * wr, C), _f32),
            pltpu.SemaphoreType.DMA,
            pltpu.SemaphoreType.DMA,
            pltpu.SemaphoreType.DMA,
        ],
    )
    def k(*refs):
        if with_resid:
            y_hbm, i_hbm, r_hbm, o_hbm = refs[:4]
        else:
            y_hbm, i_hbm, o_hbm = refs[:3]
        idxw_v, idv_v, z_v, ga_v, gb_v, acc_sh, sem_a, sem_b, sem_m = refs[-9:]
        cid = lax.axis_index("c")
        sid = lax.axis_index("s")
        wid = sid * 2 + cid

        @pl.loop(0, wr // 16)
        def _(j):
            sl = pl.ds(j * 16, 16)
            idv_v[sl] = jnp.arange(16, dtype=jnp.int32) + (j * 16 + sid * wr)

        @pl.loop(0, wr)
        def _(r):
            for cc in range(C // 16):
                z_v[r, pl.ds(cc * 16, 16)] = jnp.zeros((16,), _f32)

        def gather(kk, buf, sem):
            return pltpu.async_copy(y_hbm.at[idxw_v.at[kk]], buf, sem)

        def scat_add(buf):
            pltpu.sync_copy(buf, acc_sh.at[idv_v], add=True)

        @pl.loop(0, iters)
        def _(it):
            w = wid + NWORK * it

            @pl.when(w < nwin)
            def _():
                base = w * wr
                pltpu.async_copy(i_hbm.at[:, w, :], idxw_v, sem_m).wait()
                if with_resid:
                    pltpu.async_copy(
                        r_hbm.at[pl.ds(base, wr)],
                        acc_sh.at[pl.ds(sid * wr, wr)], sem_m).wait()
                else:
                    pltpu.async_copy(
                        z_v, acc_sh.at[pl.ds(sid * wr, wr)], sem_m).wait()
                gather(0, ga_v, sem_a)

                @pl.loop(0, (K - 1) // 2)
                def _(j):
                    gather(2 * j + 1, gb_v, sem_b)
                    pltpu.make_async_copy(y_hbm.at[idxw_v.at[0]], ga_v,
                                          sem_a).wait()
                    scat_add(ga_v)
                    gather(2 * j + 2, ga_v, sem_a)
                    pltpu.make_async_copy(y_hbm.at[idxw_v.at[0]], gb_v,
                                          sem_b).wait()
                    scat_add(gb_v)

                pltpu.make_async_copy(y_hbm.at[idxw_v.at[0]], ga_v,
                                      sem_a).wait()
                scat_add(ga_v)
                pltpu.async_copy(acc_sh.at[pl.ds(sid * wr, wr)],
                                 o_hbm.at[pl.ds(base, wr)], sem_m).wait()

    args = (yflat, idxw, resid) if with_resid else (yflat, idxw)
    return k(*args)


def _affine(s, ss, gamma, beta):
    cnt = float(NB * (C // G))
    sg = s[0].reshape(G, C // G).sum(-1)
    ssg = ss[0].reshape(G, C // G).sum(-1)
    mean = sg / cnt
    var = ssg / cnt - mean * mean
    rstd = lax.rsqrt(var + 1e-5)
    meanc = jnp.repeat(mean, C // G)
    rstdc = jnp.repeat(rstd, C // G)
    a = rstdc * gamma
    b = beta - meanc * a
    pad = jnp.zeros((7, C), _f32)
    return (jnp.concatenate([a[None, :], pad], 0),
            jnp.concatenate([b[None, :], pad], 0))


def kernel(feats, emb, gamma1, beta1, W1, b1c, We, be, gamma2, beta2, W2, b2c,
           batch_idx, nbrs):
    emb8 = jnp.concatenate([emb, jnp.zeros((8 - B, emb.shape[1]), _f32)], 0)
    bevec = jnp.broadcast_to(be[None, :], (8, C))
    z8 = jnp.zeros((8, C), _f32)
    zrow = jnp.zeros((7, C), _f32)
    b1vec = jnp.concatenate(
        [jnp.broadcast_to(b1c[None, :], (8, C)) if k == 13 else z8
         for k in range(K)], axis=1)
    b2vec = jnp.concatenate(
        [jnp.broadcast_to(b2c[None, :], (8, C)) if k == 13 else z8
         for k in range(K)], axis=1)
    W1cat = W1.transpose(1, 0, 2).reshape(C, K * C)
    W2cat = W2.transpose(1, 0, 2).reshape(C, K * C)

    eo = _emb_mlp(emb8, We, bevec)
    kvec = jnp.arange(K, dtype=jnp.int32)[:, None]
    spread = NB + (jnp.arange(NB, dtype=jnp.int32) % TM)[None, :]

    outs = []
    for b in range(B):
        fb = feats[b * NB:(b + 1) * NB]
        vb = nbrs[:, b * NB:(b + 1) * NB]
        idxwb = (jnp.where(vb < SENT, vb - b * NB, spread) * K + kvec)
        idxwb = idxwb.reshape(K, NWINB, WR)
        eob = jnp.concatenate([eo[b:b + 1], zrow], 0)

        s1, ss1 = _stats(fb, z8)
        a1, b1 = _affine(s1, ss1, gamma1, beta1)
        y1 = _mm27(fb, z8, a1, b1, W1cat, b1vec)
        h1 = _sc_gather_sum(y1, idxwb, None)

        s2, ss2 = _stats(h1, eob)
        a2, b2 = _affine(s2, ss2, gamma2, beta2)
        y2 = _mm27(h1, eob, a2, b2, W2cat, b2vec)
        outs.append(_sc_gather_sum(y2, idxwb, fb))

    return jnp.concatenate(outs, 0)
